# Initial kernel scaffold; baseline (speedup 1.0000x reference)
import jax
import jax.numpy as jnp
from jax.experimental import pallas as pl


def kernel(f_atoms, f_bonds, a2b, a2a, b2a, b2revb, W_h_q, W_h_k, W_h_v):
    raise NotImplementedError("write your pallas kernel here")


# speedup vs baseline: 1.1820x; 1.1820x over previous
import functools

import jax
import jax.numpy as jnp
from jax import lax
from jax.experimental import pallas as pl
from jax.experimental.pallas import tpu as pltpu
from jax.experimental.pallas import tpu_sc as plsc

N_ATOMS = 10000
N_BONDS = 320000
HIDDEN = 128
MAX_NB = 32
DEPTH = 6

NW = 32
ATOMS_PAD = 10240
ATOMS_PER_W = ATOMS_PAD // NW
A_CHUNK = 4
A_NCHUNK = ATOMS_PER_W // A_CHUNK
BONDS_PER_W = N_BONDS // NW
B_CHUNK = 80
B_NCHUNK = BONDS_PER_W // B_CHUNK


def _make_nei_kernel(width):
    ncg = width // 16
    mesh = plsc.VectorSubcoreMesh(
        core_axis_name="c", subcore_axis_name="s", num_cores=2, num_subcores=16)

    @functools.partial(
        pl.kernel,
        out_type=jax.ShapeDtypeStruct((ATOMS_PAD, width), jnp.float32),
        mesh=mesh,
        scratch_types=[
            pltpu.VMEM((A_NCHUNK, 128), jnp.int32),
            pltpu.VMEM((A_CHUNK * MAX_NB, width), jnp.float32),
            pltpu.VMEM((A_CHUNK, width), jnp.float32),
            pltpu.SemaphoreType.DMA,
        ],
    )
    def nei_kernel(msg_hbm, a2b_hbm, out_hbm, idx_v, rows_v, acc_v, sem):
        wid = lax.axis_index("s") * 2 + lax.axis_index("c")
        base_atom = wid * ATOMS_PER_W
        pltpu.sync_copy(a2b_hbm.at[wid], idx_v)

        def chunk_body(c, _):
            pltpu.async_copy(msg_hbm.at[idx_v.at[c]], rows_v, sem).wait()
            for a in range(A_CHUNK):
                def nb_body(nb, carry):
                    row = a * MAX_NB + nb
                    return tuple(
                        carry[cg] + rows_v[row, pl.ds(cg * 16, 16)]
                        for cg in range(ncg)
                    )
                acc = lax.fori_loop(
                    0, MAX_NB, nb_body,
                    tuple(jnp.zeros((16,), jnp.float32) for _ in range(ncg)),
                )
                for cg in range(ncg):
                    acc_v[a, pl.ds(cg * 16, 16)] = acc[cg]
            pltpu.sync_copy(
                acc_v, out_hbm.at[pl.ds(base_atom + c * A_CHUNK, A_CHUNK)])
            return _

        lax.fori_loop(0, A_NCHUNK, chunk_body, 0)

    return nei_kernel


def _make_comb_kernel(width):
    ncg = width // 16
    mesh = plsc.VectorSubcoreMesh(
        core_axis_name="c", subcore_axis_name="s", num_cores=2, num_subcores=16)

    @functools.partial(
        pl.kernel,
        out_type=jax.ShapeDtypeStruct((N_BONDS, width), jnp.float32),
        mesh=mesh,
        scratch_types=[
            pltpu.VMEM((B_NCHUNK, B_CHUNK), jnp.int32),
            pltpu.VMEM((B_NCHUNK, B_CHUNK), jnp.int32),
            pltpu.VMEM((B_CHUNK, width), jnp.float32),
            pltpu.VMEM((B_CHUNK, width), jnp.float32),
            pltpu.SemaphoreType.DMA,
        ],
    )
    def comb_kernel(nei_hbm, msg_hbm, b2a_hbm, b2revb_hbm, out_hbm,
                    idxa_v, idxr_v, nrows_v, mrows_v, sem):
        wid = lax.axis_index("s") * 2 + lax.axis_index("c")
        base_bond = wid * BONDS_PER_W
        pltpu.sync_copy(b2a_hbm.at[wid], idxa_v)
        pltpu.sync_copy(b2revb_hbm.at[wid], idxr_v)

        def chunk_body(c, _):
            cp1 = pltpu.async_copy(nei_hbm.at[idxa_v.at[c]], nrows_v, sem)
            cp2 = pltpu.async_copy(msg_hbm.at[idxr_v.at[c]], mrows_v, sem)
            cp1.wait()
            cp2.wait()

            def row_body(r, _):
                for cg in range(ncg):
                    sl = pl.ds(cg * 16, 16)
                    nrows_v[r, sl] = nrows_v[r, sl] - mrows_v[r, sl]
                return _

            lax.fori_loop(0, B_CHUNK, row_body, 0)
            pltpu.sync_copy(
                nrows_v, out_hbm.at[pl.ds(base_bond + c * B_CHUNK, B_CHUNK)])
            return _

        lax.fori_loop(0, B_NCHUNK, chunk_body, 0)

    return comb_kernel


_MM_ROWS = 1280


def _mm_body(x_ref, w_ref, o_ref):
    o_ref[...] = jnp.maximum(
        jnp.dot(x_ref[...], w_ref[0], preferred_element_type=jnp.float32), 0.0)


def _matmul_relu(x, wt_stack, in_width):
    if in_width == HIDDEN:
        x_map = lambda i, j: (i, 0)
    else:
        x_map = lambda i, j: (i, j)
    return pl.pallas_call(
        _mm_body,
        grid=(N_BONDS // _MM_ROWS, 3),
        in_specs=[
            pl.BlockSpec((_MM_ROWS, HIDDEN), x_map),
            pl.BlockSpec((1, HIDDEN, HIDDEN), lambda i, j: (j, 0, 0)),
        ],
        out_specs=pl.BlockSpec((_MM_ROWS, HIDDEN), lambda i, j: (i, j)),
        out_shape=jax.ShapeDtypeStruct((N_BONDS, 3 * HIDDEN), jnp.float32),
        compiler_params=pltpu.CompilerParams(
            dimension_semantics=("parallel", "arbitrary")),
    )(x, wt_stack)


_make_nei_kernel = functools.lru_cache(maxsize=None)(_make_nei_kernel)
_make_comb_kernel = functools.lru_cache(maxsize=None)(_make_comb_kernel)


def kernel(f_atoms, f_bonds, a2b, a2a, b2a, b2revb, W_h_q, W_h_k, W_h_v):
    del f_atoms, a2a
    a2b = a2b.astype(jnp.int32)
    b2a = b2a.astype(jnp.int32)
    b2revb = b2revb.astype(jnp.int32)

    a2b_pad = jnp.zeros((ATOMS_PAD, MAX_NB), jnp.int32).at[:N_ATOMS].set(a2b)
    a2b_r = a2b_pad.reshape(NW, A_NCHUNK, 128)
    b2a_r = b2a.reshape(NW, B_NCHUNK, B_CHUNK)
    b2revb_r = b2revb.reshape(NW, B_NCHUNK, B_CHUNK)
    wt_stack = jnp.stack([W_h_q.T, W_h_k.T, W_h_v.T])

    msg = f_bonds
    nei = _make_nei_kernel(HIDDEN)(msg, a2b_r)
    new = _make_comb_kernel(HIDDEN)(nei, msg, b2a_r, b2revb_r)
    msg = _matmul_relu(new, wt_stack, HIDDEN)
    for _ in range(DEPTH - 2):
        nei = _make_nei_kernel(3 * HIDDEN)(msg, a2b_r)
        new = _make_comb_kernel(3 * HIDDEN)(nei, msg, b2a_r, b2revb_r)
        msg = _matmul_relu(new, wt_stack, 3 * HIDDEN)

    return (msg[:, :HIDDEN], msg[:, HIDDEN:2 * HIDDEN], msg[:, 2 * HIDDEN:])


# speedup vs baseline: 1.4286x; 1.2086x over previous
import functools

import jax
import jax.numpy as jnp
from jax import lax
from jax.experimental import pallas as pl
from jax.experimental.pallas import tpu as pltpu
from jax.experimental.pallas import tpu_sc as plsc

N_ATOMS = 10000
N_BONDS = 320000
HIDDEN = 128
MAX_NB = 32
DEPTH = 6

NW = 32
ATOMS_PAD = 10240
ATOMS_PER_W = ATOMS_PAD // NW
A_CHUNK = 4
A_NCHUNK = ATOMS_PER_W // A_CHUNK
BONDS_PER_W = N_BONDS // NW
B_CHUNK = 40
B_NCHUNK = BONDS_PER_W // B_CHUNK


def _make_nei_kernel(width):
    ncg = width // 16
    mesh = plsc.VectorSubcoreMesh(
        core_axis_name="c", subcore_axis_name="s", num_cores=2, num_subcores=16)

    @functools.partial(
        pl.kernel,
        out_type=jax.ShapeDtypeStruct((ATOMS_PAD, width), jnp.float32),
        mesh=mesh,
        scratch_types=[
            pltpu.VMEM((A_NCHUNK, 128), jnp.int32),
            pltpu.VMEM((A_CHUNK * MAX_NB, width), jnp.float32),
            pltpu.VMEM((A_CHUNK * MAX_NB, width), jnp.float32),
            pltpu.VMEM((A_CHUNK, width), jnp.float32),
            pltpu.SemaphoreType.DMA,
            pltpu.SemaphoreType.DMA,
        ],
    )
    def nei_kernel(msg_hbm, a2b_hbm, out_hbm, idx_v, rows0_v, rows1_v, acc_v,
                   sem0, sem1):
        wid = lax.axis_index("s") * 2 + lax.axis_index("c")
        base_atom = wid * ATOMS_PER_W
        pltpu.sync_copy(a2b_hbm.at[wid], idx_v)

        def start(c, rows_v, sem):
            pltpu.async_copy(msg_hbm.at[idx_v.at[c]], rows_v, sem)

        def wait(c, rows_v, sem):
            pltpu.make_async_copy(msg_hbm.at[idx_v.at[c]], rows_v, sem).wait()

        def compute(c, rows_v):
            for a in range(A_CHUNK):
                def nb_body(nb, carry):
                    row = a * MAX_NB + nb
                    return tuple(
                        carry[cg] + rows_v[row, pl.ds(cg * 16, 16)]
                        for cg in range(ncg)
                    )
                acc = lax.fori_loop(
                    0, MAX_NB, nb_body,
                    tuple(jnp.zeros((16,), jnp.float32) for _ in range(ncg)),
                )
                for cg in range(ncg):
                    acc_v[a, pl.ds(cg * 16, 16)] = acc[cg]
            pltpu.sync_copy(
                acc_v, out_hbm.at[pl.ds(base_atom + c * A_CHUNK, A_CHUNK)])

        start(0, rows0_v, sem0)

        def pair_body(c2, _):
            c = c2 * 2
            wait(c, rows0_v, sem0)
            start(c + 1, rows1_v, sem1)
            compute(c, rows0_v)
            wait(c + 1, rows1_v, sem1)

            @pl.when(c2 + 1 < A_NCHUNK // 2)
            def _start_next():
                start(c + 2, rows0_v, sem0)

            compute(c + 1, rows1_v)
            return _

        lax.fori_loop(0, A_NCHUNK // 2, pair_body, 0)

    return nei_kernel


def _make_comb_kernel(width):
    ncg = width // 16
    mesh = plsc.VectorSubcoreMesh(
        core_axis_name="c", subcore_axis_name="s", num_cores=2, num_subcores=16)

    @functools.partial(
        pl.kernel,
        out_type=jax.ShapeDtypeStruct((N_BONDS, width), jnp.float32),
        mesh=mesh,
        scratch_types=[
            pltpu.VMEM((B_NCHUNK, B_CHUNK), jnp.int32),
            pltpu.VMEM((B_NCHUNK, B_CHUNK), jnp.int32),
            pltpu.VMEM((B_CHUNK, width), jnp.float32),
            pltpu.VMEM((B_CHUNK, width), jnp.float32),
            pltpu.VMEM((B_CHUNK, width), jnp.float32),
            pltpu.VMEM((B_CHUNK, width), jnp.float32),
            pltpu.SemaphoreType.DMA,
            pltpu.SemaphoreType.DMA,
        ],
    )
    def comb_kernel(nei_hbm, msg_hbm, b2a_hbm, b2revb_hbm, out_hbm,
                    idxa_v, idxr_v, nrows0_v, nrows1_v, mrows0_v, mrows1_v,
                    sem0, sem1):
        wid = lax.axis_index("s") * 2 + lax.axis_index("c")
        base_bond = wid * BONDS_PER_W
        pltpu.sync_copy(b2a_hbm.at[wid], idxa_v)
        pltpu.sync_copy(b2revb_hbm.at[wid], idxr_v)

        def start(c, nrows_v, mrows_v, sem):
            pltpu.async_copy(nei_hbm.at[idxa_v.at[c]], nrows_v, sem)
            pltpu.async_copy(msg_hbm.at[idxr_v.at[c]], mrows_v, sem)

        def wait(c, nrows_v, mrows_v, sem):
            pltpu.make_async_copy(nei_hbm.at[idxa_v.at[c]], nrows_v, sem).wait()
            pltpu.make_async_copy(msg_hbm.at[idxr_v.at[c]], mrows_v, sem).wait()

        def compute(c, nrows_v, mrows_v):
            def row_body(r, _):
                for cg in range(ncg):
                    sl = pl.ds(cg * 16, 16)
                    nrows_v[r, sl] = nrows_v[r, sl] - mrows_v[r, sl]
                return _

            lax.fori_loop(0, B_CHUNK, row_body, 0)
            pltpu.sync_copy(
                nrows_v, out_hbm.at[pl.ds(base_bond + c * B_CHUNK, B_CHUNK)])

        start(0, nrows0_v, mrows0_v, sem0)

        def pair_body(c2, _):
            c = c2 * 2
            wait(c, nrows0_v, mrows0_v, sem0)
            start(c + 1, nrows1_v, mrows1_v, sem1)
            compute(c, nrows0_v, mrows0_v)
            wait(c + 1, nrows1_v, mrows1_v, sem1)

            @pl.when(c2 + 1 < B_NCHUNK // 2)
            def _start_next():
                start(c + 2, nrows0_v, mrows0_v, sem0)

            compute(c + 1, nrows1_v, mrows1_v)
            return _

        lax.fori_loop(0, B_NCHUNK // 2, pair_body, 0)

    return comb_kernel


_MM_ROWS = 1280


def _mm_body(x_ref, w_ref, o_ref):
    o_ref[...] = jnp.maximum(
        jnp.dot(x_ref[...], w_ref[0], preferred_element_type=jnp.float32), 0.0)


def _matmul_relu(x, wt_stack, in_width):
    if in_width == HIDDEN:
        x_map = lambda i, j: (i, 0)
    else:
        x_map = lambda i, j: (i, j)
    return pl.pallas_call(
        _mm_body,
        grid=(N_BONDS // _MM_ROWS, 3),
        in_specs=[
            pl.BlockSpec((_MM_ROWS, HIDDEN), x_map),
            pl.BlockSpec((1, HIDDEN, HIDDEN), lambda i, j: (j, 0, 0)),
        ],
        out_specs=pl.BlockSpec((_MM_ROWS, HIDDEN), lambda i, j: (i, j)),
        out_shape=jax.ShapeDtypeStruct((N_BONDS, 3 * HIDDEN), jnp.float32),
        compiler_params=pltpu.CompilerParams(
            dimension_semantics=("parallel", "arbitrary")),
    )(x, wt_stack)


_make_nei_kernel = functools.lru_cache(maxsize=None)(_make_nei_kernel)
_make_comb_kernel = functools.lru_cache(maxsize=None)(_make_comb_kernel)


def kernel(f_atoms, f_bonds, a2b, a2a, b2a, b2revb, W_h_q, W_h_k, W_h_v):
    del f_atoms, a2a
    a2b = a2b.astype(jnp.int32)
    b2a = b2a.astype(jnp.int32)
    b2revb = b2revb.astype(jnp.int32)

    a2b_pad = jnp.zeros((ATOMS_PAD, MAX_NB), jnp.int32).at[:N_ATOMS].set(a2b)
    a2b_r = a2b_pad.reshape(NW, A_NCHUNK, 128)
    b2a_r = b2a.reshape(NW, B_NCHUNK, B_CHUNK)
    b2revb_r = b2revb.reshape(NW, B_NCHUNK, B_CHUNK)
    wt_stack = jnp.stack([W_h_q.T, W_h_k.T, W_h_v.T])

    msg = f_bonds
    nei = _make_nei_kernel(HIDDEN)(msg, a2b_r)
    new = _make_comb_kernel(HIDDEN)(nei, msg, b2a_r, b2revb_r)
    msg = _matmul_relu(new, wt_stack, HIDDEN)
    for _ in range(DEPTH - 2):
        nei = _make_nei_kernel(3 * HIDDEN)(msg, a2b_r)
        new = _make_comb_kernel(3 * HIDDEN)(nei, msg, b2a_r, b2revb_r)
        msg = _matmul_relu(new, wt_stack, 3 * HIDDEN)

    return (msg[:, :HIDDEN], msg[:, HIDDEN:2 * HIDDEN], msg[:, 2 * HIDDEN:])


# speedup vs baseline: 1.4613x; 1.0229x over previous
import functools

import jax
import jax.numpy as jnp
from jax import lax
from jax.experimental import pallas as pl
from jax.experimental.pallas import tpu as pltpu
from jax.experimental.pallas import tpu_sc as plsc

N_ATOMS = 10000
N_BONDS = 320000
HIDDEN = 128
MAX_NB = 32
DEPTH = 6

NW = 32
ATOMS_PAD = 10240
ATOMS_PER_W = ATOMS_PAD // NW
A_CHUNK = 2
A_NCHUNK = ATOMS_PER_W // A_CHUNK
A_NBUF = 4
BONDS_PER_W = N_BONDS // NW
B_CHUNK = 40
B_NCHUNK = BONDS_PER_W // B_CHUNK


def _make_nei_kernel(width):
    ncg = width // 16
    mesh = plsc.VectorSubcoreMesh(
        core_axis_name="c", subcore_axis_name="s", num_cores=2, num_subcores=16)

    @functools.partial(
        pl.kernel,
        out_type=jax.ShapeDtypeStruct((ATOMS_PAD, width), jnp.float32),
        mesh=mesh,
        scratch_types=(
            [pltpu.VMEM((A_NCHUNK, 64), jnp.int32)]
            + [pltpu.VMEM((A_CHUNK * MAX_NB, width), jnp.float32)
               for _ in range(A_NBUF)]
            + [pltpu.VMEM((A_CHUNK, width), jnp.float32)
               for _ in range(A_NBUF)]
            + [pltpu.SemaphoreType.DMA for _ in range(A_NBUF)]
            + [pltpu.SemaphoreType.DMA for _ in range(A_NBUF)]
        ),
    )
    def nei_kernel(msg_hbm, a2b_hbm, out_hbm, idx_v, *bufs):
        rows = bufs[0:A_NBUF]
        accs = bufs[A_NBUF:2 * A_NBUF]
        gsems = bufs[2 * A_NBUF:3 * A_NBUF]
        wsems = bufs[3 * A_NBUF:4 * A_NBUF]
        wid = lax.axis_index("s") * 2 + lax.axis_index("c")
        base_atom = wid * ATOMS_PER_W
        pltpu.sync_copy(a2b_hbm.at[wid], idx_v)

        def start(c, k):
            pltpu.async_copy(msg_hbm.at[idx_v.at[c]], rows[k], gsems[k])

        def wait(c, k):
            pltpu.make_async_copy(
                msg_hbm.at[idx_v.at[c]], rows[k], gsems[k]).wait()

        def out_slot(c):
            return out_hbm.at[pl.ds(base_atom + c * A_CHUNK, A_CHUNK)]

        def compute(c, k):
            rows_v, acc_v = rows[k], accs[k]
            for a in range(A_CHUNK):
                def nb_body(q, carry):
                    out = carry
                    for u in range(4):
                        row = a * MAX_NB + q * 4 + u
                        out = tuple(
                            out[cg] + rows_v[row, pl.ds(cg * 16, 16)]
                            for cg in range(ncg)
                        )
                    return out
                acc = lax.fori_loop(
                    0, MAX_NB // 4, nb_body,
                    tuple(jnp.zeros((16,), jnp.float32) for _ in range(ncg)),
                )
                for cg in range(ncg):
                    acc_v[a, pl.ds(cg * 16, 16)] = acc[cg]
            pltpu.async_copy(acc_v, out_slot(c), wsems[k])

        for k in range(A_NBUF):
            start(k, k)

        def ring_body(c4, _):
            c = c4 * A_NBUF
            for k in range(A_NBUF):
                wait(c + k, k)

                @pl.when(c4 > 0)
                def _drain():
                    pltpu.make_async_copy(
                        accs[k], out_slot(c + k - A_NBUF), wsems[k]).wait()

                compute(c + k, k)

                @pl.when(c + k + A_NBUF < A_NCHUNK)
                def _next():
                    start(c + k + A_NBUF, k)

            return _

        lax.fori_loop(0, A_NCHUNK // A_NBUF, ring_body, 0)
        for k in range(A_NBUF):
            pltpu.make_async_copy(
                accs[k], out_slot(A_NCHUNK - A_NBUF + k), wsems[k]).wait()

    return nei_kernel


def _make_comb_kernel(width):
    ncg = width // 16
    mesh = plsc.VectorSubcoreMesh(
        core_axis_name="c", subcore_axis_name="s", num_cores=2, num_subcores=16)

    @functools.partial(
        pl.kernel,
        out_type=jax.ShapeDtypeStruct((N_BONDS, width), jnp.float32),
        mesh=mesh,
        scratch_types=[
            pltpu.VMEM((B_NCHUNK, B_CHUNK), jnp.int32),
            pltpu.VMEM((B_NCHUNK, B_CHUNK), jnp.int32),
            pltpu.VMEM((B_CHUNK, width), jnp.float32),
            pltpu.VMEM((B_CHUNK, width), jnp.float32),
            pltpu.VMEM((B_CHUNK, width), jnp.float32),
            pltpu.VMEM((B_CHUNK, width), jnp.float32),
            pltpu.SemaphoreType.DMA,
            pltpu.SemaphoreType.DMA,
        ],
    )
    def comb_kernel(nei_hbm, msg_hbm, b2a_hbm, b2revb_hbm, out_hbm,
                    idxa_v, idxr_v, nrows0_v, nrows1_v, mrows0_v, mrows1_v,
                    sem0, sem1):
        wid = lax.axis_index("s") * 2 + lax.axis_index("c")
        base_bond = wid * BONDS_PER_W
        pltpu.sync_copy(b2a_hbm.at[wid], idxa_v)
        pltpu.sync_copy(b2revb_hbm.at[wid], idxr_v)

        def start(c, nrows_v, mrows_v, sem):
            pltpu.async_copy(nei_hbm.at[idxa_v.at[c]], nrows_v, sem)
            pltpu.async_copy(msg_hbm.at[idxr_v.at[c]], mrows_v, sem)

        def wait(c, nrows_v, mrows_v, sem):
            pltpu.make_async_copy(nei_hbm.at[idxa_v.at[c]], nrows_v, sem).wait()
            pltpu.make_async_copy(msg_hbm.at[idxr_v.at[c]], mrows_v, sem).wait()

        def compute(c, nrows_v, mrows_v):
            def row_body(r, _):
                for cg in range(ncg):
                    sl = pl.ds(cg * 16, 16)
                    nrows_v[r, sl] = nrows_v[r, sl] - mrows_v[r, sl]
                return _

            lax.fori_loop(0, B_CHUNK, row_body, 0)
            pltpu.sync_copy(
                nrows_v, out_hbm.at[pl.ds(base_bond + c * B_CHUNK, B_CHUNK)])

        start(0, nrows0_v, mrows0_v, sem0)

        def pair_body(c2, _):
            c = c2 * 2
            wait(c, nrows0_v, mrows0_v, sem0)
            start(c + 1, nrows1_v, mrows1_v, sem1)
            compute(c, nrows0_v, mrows0_v)
            wait(c + 1, nrows1_v, mrows1_v, sem1)

            @pl.when(c2 + 1 < B_NCHUNK // 2)
            def _start_next():
                start(c + 2, nrows0_v, mrows0_v, sem0)

            compute(c + 1, nrows1_v, mrows1_v)
            return _

        lax.fori_loop(0, B_NCHUNK // 2, pair_body, 0)

    return comb_kernel


_MM_ROWS = 1280


def _mm_body(x_ref, w_ref, o_ref):
    o_ref[...] = jnp.maximum(
        jnp.dot(x_ref[...], w_ref[0], preferred_element_type=jnp.float32), 0.0)


def _matmul_relu(x, wt_stack, in_width):
    if in_width == HIDDEN:
        x_map = lambda i, j: (i, 0)
    else:
        x_map = lambda i, j: (i, j)
    return pl.pallas_call(
        _mm_body,
        grid=(N_BONDS // _MM_ROWS, 3),
        in_specs=[
            pl.BlockSpec((_MM_ROWS, HIDDEN), x_map),
            pl.BlockSpec((1, HIDDEN, HIDDEN), lambda i, j: (j, 0, 0)),
        ],
        out_specs=pl.BlockSpec((_MM_ROWS, HIDDEN), lambda i, j: (i, j)),
        out_shape=jax.ShapeDtypeStruct((N_BONDS, 3 * HIDDEN), jnp.float32),
        compiler_params=pltpu.CompilerParams(
            dimension_semantics=("parallel", "arbitrary")),
    )(x, wt_stack)


_make_nei_kernel = functools.lru_cache(maxsize=None)(_make_nei_kernel)
_make_comb_kernel = functools.lru_cache(maxsize=None)(_make_comb_kernel)


def kernel(f_atoms, f_bonds, a2b, a2a, b2a, b2revb, W_h_q, W_h_k, W_h_v):
    del f_atoms, a2a
    a2b = a2b.astype(jnp.int32)
    b2a = b2a.astype(jnp.int32)
    b2revb = b2revb.astype(jnp.int32)

    a2b_pad = jnp.zeros((ATOMS_PAD, MAX_NB), jnp.int32).at[:N_ATOMS].set(a2b)
    a2b_r = a2b_pad.reshape(NW, A_NCHUNK, A_CHUNK * MAX_NB)
    b2a_r = b2a.reshape(NW, B_NCHUNK, B_CHUNK)
    b2revb_r = b2revb.reshape(NW, B_NCHUNK, B_CHUNK)
    wt_stack = jnp.stack([W_h_q.T, W_h_k.T, W_h_v.T])

    msg = f_bonds
    nei = _make_nei_kernel(HIDDEN)(msg, a2b_r)
    new = _make_comb_kernel(HIDDEN)(nei, msg, b2a_r, b2revb_r)
    msg = _matmul_relu(new, wt_stack, HIDDEN)
    for _ in range(DEPTH - 2):
        nei = _make_nei_kernel(3 * HIDDEN)(msg, a2b_r)
        new = _make_comb_kernel(3 * HIDDEN)(nei, msg, b2a_r, b2revb_r)
        msg = _matmul_relu(new, wt_stack, 3 * HIDDEN)

    return (msg[:, :HIDDEN], msg[:, HIDDEN:2 * HIDDEN], msg[:, 2 * HIDDEN:])


# speedup vs baseline: 1.4717x; 1.0072x over previous
import functools

import jax
import jax.numpy as jnp
from jax import lax
from jax.experimental import pallas as pl
from jax.experimental.pallas import tpu as pltpu
from jax.experimental.pallas import tpu_sc as plsc

N_ATOMS = 10000
N_BONDS = 320000
HIDDEN = 128
MAX_NB = 32
DEPTH = 6

NW = 32
ATOMS_PAD = 10240
ATOMS_PER_W = ATOMS_PAD // NW
A_CHUNK = 1
A_NCHUNK = ATOMS_PER_W // A_CHUNK
A_NBUF = 4
BONDS_PER_W = N_BONDS // NW
B_CHUNK = 40
B_NCHUNK = BONDS_PER_W // B_CHUNK


def _make_nei_kernel(width):
    ncg = width // 16
    mesh = plsc.VectorSubcoreMesh(
        core_axis_name="c", subcore_axis_name="s", num_cores=2, num_subcores=16)

    @functools.partial(
        pl.kernel,
        out_type=jax.ShapeDtypeStruct((ATOMS_PAD, width), jnp.float32),
        mesh=mesh,
        scratch_types=(
            [pltpu.VMEM((A_NCHUNK, A_CHUNK * MAX_NB), jnp.int32)]
            + [pltpu.VMEM((A_CHUNK * MAX_NB, width), jnp.float32)
               for _ in range(A_NBUF)]
            + [pltpu.VMEM((A_CHUNK, width), jnp.float32)
               for _ in range(A_NBUF)]
            + [pltpu.SemaphoreType.DMA for _ in range(A_NBUF)]
            + [pltpu.SemaphoreType.DMA for _ in range(A_NBUF)]
        ),
    )
    def nei_kernel(msg_hbm, a2b_hbm, out_hbm, idx_v, *bufs):
        rows = bufs[0:A_NBUF]
        accs = bufs[A_NBUF:2 * A_NBUF]
        gsems = bufs[2 * A_NBUF:3 * A_NBUF]
        wsems = bufs[3 * A_NBUF:4 * A_NBUF]
        wid = lax.axis_index("s") * 2 + lax.axis_index("c")
        base_atom = wid * ATOMS_PER_W
        pltpu.sync_copy(a2b_hbm.at[wid], idx_v)

        def start(c, k):
            pltpu.async_copy(msg_hbm.at[idx_v.at[c]], rows[k], gsems[k])

        def wait(c, k):
            pltpu.make_async_copy(
                msg_hbm.at[idx_v.at[c]], rows[k], gsems[k]).wait()

        def out_slot(c):
            return out_hbm.at[pl.ds(base_atom + c * A_CHUNK, A_CHUNK)]

        def compute(c, k):
            rows_v, acc_v = rows[k], accs[k]
            for a in range(A_CHUNK):
                def nb_body(q, carry):
                    out = carry
                    for u in range(4):
                        row = a * MAX_NB + q * 4 + u
                        out = tuple(
                            out[cg] + rows_v[row, pl.ds(cg * 16, 16)]
                            for cg in range(ncg)
                        )
                    return out
                acc = lax.fori_loop(
                    0, MAX_NB // 4, nb_body,
                    tuple(jnp.zeros((16,), jnp.float32) for _ in range(ncg)),
                )
                for cg in range(ncg):
                    acc_v[a, pl.ds(cg * 16, 16)] = acc[cg]
            pltpu.async_copy(acc_v, out_slot(c), wsems[k])

        for k in range(A_NBUF):
            start(k, k)

        def ring_body(c4, _):
            c = c4 * A_NBUF
            for k in range(A_NBUF):
                wait(c + k, k)

                @pl.when(c4 > 0)
                def _drain():
                    pltpu.make_async_copy(
                        accs[k], out_slot(c + k - A_NBUF), wsems[k]).wait()

                compute(c + k, k)

                @pl.when(c + k + A_NBUF < A_NCHUNK)
                def _next():
                    start(c + k + A_NBUF, k)

            return _

        lax.fori_loop(0, A_NCHUNK // A_NBUF, ring_body, 0)
        for k in range(A_NBUF):
            pltpu.make_async_copy(
                accs[k], out_slot(A_NCHUNK - A_NBUF + k), wsems[k]).wait()

    return nei_kernel


def _make_comb_kernel(width):
    ncg = width // 16
    mesh = plsc.VectorSubcoreMesh(
        core_axis_name="c", subcore_axis_name="s", num_cores=2, num_subcores=16)

    @functools.partial(
        pl.kernel,
        out_type=jax.ShapeDtypeStruct((N_BONDS, width), jnp.float32),
        mesh=mesh,
        scratch_types=[
            pltpu.VMEM((B_NCHUNK, B_CHUNK), jnp.int32),
            pltpu.VMEM((B_NCHUNK, B_CHUNK), jnp.int32),
            pltpu.VMEM((B_CHUNK, width), jnp.float32),
            pltpu.VMEM((B_CHUNK, width), jnp.float32),
            pltpu.VMEM((B_CHUNK, width), jnp.float32),
            pltpu.VMEM((B_CHUNK, width), jnp.float32),
            pltpu.SemaphoreType.DMA,
            pltpu.SemaphoreType.DMA,
        ],
    )
    def comb_kernel(nei_hbm, msg_hbm, b2a_hbm, b2revb_hbm, out_hbm,
                    idxa_v, idxr_v, nrows0_v, nrows1_v, mrows0_v, mrows1_v,
                    sem0, sem1):
        wid = lax.axis_index("s") * 2 + lax.axis_index("c")
        base_bond = wid * BONDS_PER_W
        pltpu.sync_copy(b2a_hbm.at[wid], idxa_v)
        pltpu.sync_copy(b2revb_hbm.at[wid], idxr_v)

        def start(c, nrows_v, mrows_v, sem):
            pltpu.async_copy(nei_hbm.at[idxa_v.at[c]], nrows_v, sem)
            pltpu.async_copy(msg_hbm.at[idxr_v.at[c]], mrows_v, sem)

        def wait(c, nrows_v, mrows_v, sem):
            pltpu.make_async_copy(nei_hbm.at[idxa_v.at[c]], nrows_v, sem).wait()
            pltpu.make_async_copy(msg_hbm.at[idxr_v.at[c]], mrows_v, sem).wait()

        def compute(c, nrows_v, mrows_v):
            def row_body(r, _):
                for cg in range(ncg):
                    sl = pl.ds(cg * 16, 16)
                    nrows_v[r, sl] = nrows_v[r, sl] - mrows_v[r, sl]
                return _

            lax.fori_loop(0, B_CHUNK, row_body, 0)
            pltpu.sync_copy(
                nrows_v, out_hbm.at[pl.ds(base_bond + c * B_CHUNK, B_CHUNK)])

        start(0, nrows0_v, mrows0_v, sem0)

        def pair_body(c2, _):
            c = c2 * 2
            wait(c, nrows0_v, mrows0_v, sem0)
            start(c + 1, nrows1_v, mrows1_v, sem1)
            compute(c, nrows0_v, mrows0_v)
            wait(c + 1, nrows1_v, mrows1_v, sem1)

            @pl.when(c2 + 1 < B_NCHUNK // 2)
            def _start_next():
                start(c + 2, nrows0_v, mrows0_v, sem0)

            compute(c + 1, nrows1_v, mrows1_v)
            return _

        lax.fori_loop(0, B_NCHUNK // 2, pair_body, 0)

    return comb_kernel


_MM_ROWS = 1280


def _mm_body(x_ref, w_ref, o_ref):
    o_ref[...] = jnp.maximum(
        jnp.dot(x_ref[...], w_ref[0], preferred_element_type=jnp.float32), 0.0)


def _matmul_relu(x, wt_stack, in_width):
    if in_width == HIDDEN:
        x_map = lambda i, j: (i, 0)
    else:
        x_map = lambda i, j: (i, j)
    return pl.pallas_call(
        _mm_body,
        grid=(N_BONDS // _MM_ROWS, 3),
        in_specs=[
            pl.BlockSpec((_MM_ROWS, HIDDEN), x_map),
            pl.BlockSpec((1, HIDDEN, HIDDEN), lambda i, j: (j, 0, 0)),
        ],
        out_specs=pl.BlockSpec((_MM_ROWS, HIDDEN), lambda i, j: (i, j)),
        out_shape=jax.ShapeDtypeStruct((N_BONDS, 3 * HIDDEN), jnp.float32),
        compiler_params=pltpu.CompilerParams(
            dimension_semantics=("parallel", "arbitrary")),
    )(x, wt_stack)


_make_nei_kernel = functools.lru_cache(maxsize=None)(_make_nei_kernel)
_make_comb_kernel = functools.lru_cache(maxsize=None)(_make_comb_kernel)


def kernel(f_atoms, f_bonds, a2b, a2a, b2a, b2revb, W_h_q, W_h_k, W_h_v):
    del f_atoms, a2a
    a2b = a2b.astype(jnp.int32)
    b2a = b2a.astype(jnp.int32)
    b2revb = b2revb.astype(jnp.int32)

    a2b_pad = jnp.zeros((ATOMS_PAD, MAX_NB), jnp.int32).at[:N_ATOMS].set(a2b)
    a2b_r = a2b_pad.reshape(NW, A_NCHUNK, A_CHUNK * MAX_NB)
    b2a_r = b2a.reshape(NW, B_NCHUNK, B_CHUNK)
    b2revb_r = b2revb.reshape(NW, B_NCHUNK, B_CHUNK)
    wt_stack = jnp.stack([W_h_q.T, W_h_k.T, W_h_v.T])

    msg = f_bonds
    nei = _make_nei_kernel(HIDDEN)(msg, a2b_r)
    new = _make_comb_kernel(HIDDEN)(nei, msg, b2a_r, b2revb_r)
    msg = _matmul_relu(new, wt_stack, HIDDEN)
    for _ in range(DEPTH - 2):
        nei = _make_nei_kernel(3 * HIDDEN)(msg, a2b_r)
        new = _make_comb_kernel(3 * HIDDEN)(nei, msg, b2a_r, b2revb_r)
        msg = _matmul_relu(new, wt_stack, 3 * HIDDEN)

    return (msg[:, :HIDDEN], msg[:, HIDDEN:2 * HIDDEN], msg[:, 2 * HIDDEN:])
